# speedup vs baseline: 3.5661x; 3.5661x over previous
import functools

import jax
import jax.numpy as jnp
from jax import lax
from jax.experimental import pallas as pl
from jax.experimental.pallas import tpu as pltpu
from jax.experimental.pallas import tpu_sc as plsc

NC = 2
NS = 16
NW = NC * NS

EMBED = 128
TOTAL = 4096 * 50
PER_W = TOTAL // NW
CHUNK = 128
NCH = PER_W // CHUNK
NBUF = 2


def _emb_body(src_idx, tgt_idx, src_tab, tgt_tab, src_out, tgt_out,
              idx_v, buf0, buf1, gsem, ssem):
    wid = lax.axis_index("s") * NC + lax.axis_index("c")
    bufs = (buf0, buf1)
    out_base = wid * PER_W

    for idx_hbm, tab_hbm, out_hbm in (
        (src_idx, src_tab, src_out),
        (tgt_idx, tgt_tab, tgt_out),
    ):
        pltpu.sync_copy(idx_hbm.at[wid], idx_v)

        for b in range(NBUF):
            pltpu.async_copy(tab_hbm.at[idx_v.at[b]], bufs[b], gsem)

        @pl.loop(0, NCH - NBUF, step=NBUF)
        def _(g):
            for b in range(NBUF):
                j = g + b
                pltpu.make_async_copy(
                    tab_hbm.at[idx_v.at[j]], bufs[b], gsem).wait()
                dst = out_hbm.at[pl.ds(out_base + j * CHUNK, CHUNK)]
                pltpu.async_copy(bufs[b], dst, ssem)
                pltpu.make_async_copy(bufs[b], dst, ssem).wait()
                pltpu.async_copy(
                    tab_hbm.at[idx_v.at[j + NBUF]], bufs[b], gsem)

        for b in range(NBUF):
            j = NCH - NBUF + b
            pltpu.make_async_copy(
                tab_hbm.at[idx_v.at[j]], bufs[b], gsem).wait()
            dst = out_hbm.at[pl.ds(out_base + j * CHUNK, CHUNK)]
            pltpu.async_copy(bufs[b], dst, ssem)
            pltpu.make_async_copy(bufs[b], dst, ssem).wait()


@jax.jit
def _emb(src_idx, tgt_idx, src_tab, tgt_tab):
    mesh = plsc.VectorSubcoreMesh(
        core_axis_name="c", subcore_axis_name="s",
        num_cores=NC, num_subcores=NS)
    f = pl.kernel(
        _emb_body,
        out_type=[
            jax.ShapeDtypeStruct((TOTAL, EMBED), jnp.float32),
            jax.ShapeDtypeStruct((TOTAL, EMBED), jnp.float32),
        ],
        mesh=mesh,
        scratch_types=[
            pltpu.VMEM((NCH, CHUNK), jnp.int32),
            pltpu.VMEM((CHUNK, EMBED), jnp.float32),
            pltpu.VMEM((CHUNK, EMBED), jnp.float32),
            pltpu.SemaphoreType.DMA,
            pltpu.SemaphoreType.DMA,
        ],
    )
    return f(src_idx, tgt_idx, src_tab, tgt_tab)


def kernel(src_indices, tgt_indices, src_table, tgt_table):
    B, L = src_indices.shape
    si = src_indices.reshape(NW, NCH, CHUNK)
    ti = tgt_indices.reshape(NW, NCH, CHUNK)
    src_out, tgt_out = _emb(si, ti, src_table, tgt_table)
    return (src_out.reshape(B, L, EMBED), tgt_out.reshape(B, L, EMBED))


# speedup vs baseline: 3.5875x; 1.0060x over previous
import jax
import jax.numpy as jnp
from jax import lax
from jax.experimental import pallas as pl
from jax.experimental.pallas import tpu as pltpu
from jax.experimental.pallas import tpu_sc as plsc

NC = 2
NS = 16
NW = NC * NS

EMBED = 128
TOTAL = 4096 * 50
PER_W = TOTAL // NW
CHUNK = 128
NCH = PER_W // CHUNK
NBUF = 6
DEPTH = 3


def _emb_body(src_idx, tgt_idx, src_tab, tgt_tab, src_out, tgt_out,
              idx_v, b0, b1, b2, b3, b4, b5, gsem, ssem):
    wid = lax.axis_index("s") * NC + lax.axis_index("c")
    bufs = (b0, b1, b2, b3, b4, b5)
    out_base = wid * PER_W

    for idx_hbm, tab_hbm, out_hbm in (
        (src_idx, src_tab, src_out),
        (tgt_idx, tgt_tab, tgt_out),
    ):
        pltpu.sync_copy(idx_hbm.at[wid], idx_v)

        def gstart(j, b):
            pltpu.async_copy(tab_hbm.at[idx_v.at[j]], bufs[b], gsem)

        def body(j, b, do_swait, do_gstart):
            pltpu.make_async_copy(
                tab_hbm.at[idx_v.at[j]], bufs[b], gsem).wait()
            dst = out_hbm.at[pl.ds(out_base + j * CHUNK, CHUNK)]
            pltpu.async_copy(bufs[b], dst, ssem)
            if do_swait:
                pltpu.make_async_copy(bufs[b], dst, ssem).wait()
            if do_gstart:
                gstart(j + DEPTH, (b + DEPTH) % NBUF)

        for j in range(DEPTH):
            gstart(j, j % NBUF)

        for j in range(DEPTH):
            body(j, j % NBUF, do_swait=False, do_gstart=True)

        steady = ((NCH - 2 * DEPTH + 1) // NBUF) * NBUF

        @pl.loop(DEPTH, DEPTH + steady, step=NBUF)
        def _(g):
            for off in range(NBUF):
                jj = g + off
                body(jj, (DEPTH + off) % NBUF, do_swait=True, do_gstart=True)

        for j in range(DEPTH + steady, NCH):
            body(j, j % NBUF, do_swait=True, do_gstart=(j + DEPTH < NCH))

        for b in range(DEPTH):
            pltpu.make_async_copy(
                bufs[b], out_hbm.at[pl.ds(out_base, CHUNK)], ssem).wait()


@jax.jit
def _emb(src_idx, tgt_idx, src_tab, tgt_tab):
    mesh = plsc.VectorSubcoreMesh(
        core_axis_name="c", subcore_axis_name="s",
        num_cores=NC, num_subcores=NS)
    f = pl.kernel(
        _emb_body,
        out_type=[
            jax.ShapeDtypeStruct((TOTAL, EMBED), jnp.float32),
            jax.ShapeDtypeStruct((TOTAL, EMBED), jnp.float32),
        ],
        mesh=mesh,
        scratch_types=[pltpu.VMEM((NCH, CHUNK), jnp.int32)]
        + [pltpu.VMEM((CHUNK, EMBED), jnp.float32) for _ in range(NBUF)]
        + [pltpu.SemaphoreType.DMA, pltpu.SemaphoreType.DMA],
    )
    return f(src_idx, tgt_idx, src_tab, tgt_tab)


def kernel(src_indices, tgt_indices, src_table, tgt_table):
    B, L = src_indices.shape
    si = src_indices.reshape(NW, NCH, CHUNK)
    ti = tgt_indices.reshape(NW, NCH, CHUNK)
    src_out, tgt_out = _emb(si, ti, src_table, tgt_table)
    return (src_out.reshape(B, L, EMBED), tgt_out.reshape(B, L, EMBED))


# speedup vs baseline: 6.0027x; 1.6732x over previous
import jax
import jax.numpy as jnp
from jax import lax
from jax.experimental import pallas as pl
from jax.experimental.pallas import tpu as pltpu
from jax.experimental.pallas import tpu_sc as plsc

NC = 2
NS = 16
NW = NC * NS

B = 4096
L = 50
EMBED = 128
BPW = B // NW
GRP = 4
NGRP = BPW // GRP
NBUF = 2


def _emb_body(src_idx, tgt_idx, src_tab, tgt_tab, src_out, tgt_out,
              idx_v, buf0, buf1, gsem, ssem):
    wid = lax.axis_index("s") * NC + lax.axis_index("c")
    bufs = (buf0, buf1)
    bat_base = wid * BPW

    for idx_hbm, tab_hbm, out_hbm in (
        (src_idx, src_tab, src_out),
        (tgt_idx, tgt_tab, tgt_out),
    ):
        pltpu.sync_copy(idx_hbm.at[wid], idx_v)

        def gstart(g, b):
            for k in range(GRP):
                pltpu.async_copy(
                    tab_hbm.at[idx_v.at[g * GRP + k]], bufs[b].at[k], gsem)

        def body(g, b):
            for k in range(GRP):
                pltpu.make_async_copy(
                    tab_hbm.at[idx_v.at[g * GRP + k]], bufs[b].at[k],
                    gsem).wait()
            dst = out_hbm.at[pl.ds(bat_base + g * GRP, GRP)]
            pltpu.async_copy(bufs[b], dst, ssem)
            pltpu.make_async_copy(bufs[b], dst, ssem).wait()

        for b in range(NBUF):
            gstart(b, b)

        @pl.loop(0, NGRP - NBUF, step=NBUF)
        def _(g0):
            for b in range(NBUF):
                body(g0 + b, b)
                gstart(g0 + b + NBUF, b)

        for b in range(NBUF):
            body(NGRP - NBUF + b, b)


@jax.jit
def _emb(src_idx, tgt_idx, src_tab, tgt_tab):
    mesh = plsc.VectorSubcoreMesh(
        core_axis_name="c", subcore_axis_name="s",
        num_cores=NC, num_subcores=NS)
    f = pl.kernel(
        _emb_body,
        out_type=[
            jax.ShapeDtypeStruct((B, L, EMBED), jnp.float32),
            jax.ShapeDtypeStruct((B, L, EMBED), jnp.float32),
        ],
        mesh=mesh,
        scratch_types=[pltpu.VMEM((BPW, L), jnp.int32)]
        + [pltpu.VMEM((GRP, L, EMBED), jnp.float32) for _ in range(NBUF)]
        + [pltpu.SemaphoreType.DMA, pltpu.SemaphoreType.DMA],
    )
    return f(src_idx, tgt_idx, src_tab, tgt_tab)


def kernel(src_indices, tgt_indices, src_table, tgt_table):
    si = src_indices.reshape(NW, BPW, L)
    ti = tgt_indices.reshape(NW, BPW, L)
    src_out, tgt_out = _emb(si, ti, src_table, tgt_table)
    return (src_out, tgt_out)
